# baseline (speedup 1.0000x reference)
import functools

import jax
import jax.numpy as jnp
from jax import lax
from jax.experimental import pallas as pl
from jax.experimental.pallas import tpu as pltpu
from jax.experimental.pallas import tpu_sc as plsc

_NUM_HEADS = 12
_HEAD_DIM = 64
_TOPK = 8
_D = _NUM_HEADS * _HEAD_DIM
_BLOCK_T = 512
_CHUNK = 32
_LANES = 16


def _matmul_body(x_ref, w_ref, b_ref, o_ref):
    acc = jax.lax.dot_general(
        x_ref[...], w_ref[...],
        dimension_numbers=(((1,), (1,)), ((), ())),
        preferred_element_type=jnp.float32,
    )
    o_ref[...] = acc + b_ref[...]


def _tc_matmul(x2, W, b2):
    N = x2.shape[0]
    T = _BLOCK_T
    return pl.pallas_call(
        _matmul_body,
        grid=(N // T,),
        in_specs=[
            pl.BlockSpec((T, _D), lambda i: (i, 0)),
            pl.BlockSpec((_D, _D), lambda i: (0, 0)),
            pl.BlockSpec((1, _D), lambda i: (0, 0)),
        ],
        out_specs=pl.BlockSpec((T, _D), lambda i: (i, 0)),
        out_shape=jax.ShapeDtypeStruct((N, _D), jnp.float32),
        compiler_params=pltpu.CompilerParams(
            dimension_semantics=("arbitrary",),
        ),
    )(x2, W, b2)


_GATHER_DNUMS = lax.GatherDimensionNumbers(
    offset_dims=(), collapsed_slice_dims=(0,), start_index_map=(0,))


def _allmax(v):
    lanes = lax.iota(jnp.int32, _LANES)
    for s in range(4):
        idx = lax.bitwise_xor(lanes, jnp.int32(1 << s))
        shuf = lax.gather(
            v, idx[:, None], _GATHER_DNUMS, slice_sizes=(1,),
            mode=lax.GatherScatterMode.PROMISE_IN_BOUNDS)
        v = jnp.maximum(v, shuf)
    return v


def _mask_one_token(buf, obuf, t):
    neg = jnp.float32(-jnp.inf)
    for hd in range(_NUM_HEADS):
        base = hd * _HEAD_DIM
        vs = [buf[t, pl.ds(base + k * _LANES, _LANES)] for k in range(4)]
        ws = list(vs)
        for _ in range(_TOPK - 1):
            m01 = jnp.maximum(ws[0], ws[1])
            m23 = jnp.maximum(ws[2], ws[3])
            mb = _allmax(jnp.maximum(m01, m23))
            ws = [jnp.where(w == mb, neg, w) for w in ws]
        m01 = jnp.maximum(ws[0], ws[1])
        m23 = jnp.maximum(ws[2], ws[3])
        thr = _allmax(jnp.maximum(m01, m23))
        for k in range(4):
            obuf[t, pl.ds(base + k * _LANES, _LANES)] = jnp.where(
                vs[k] >= thr, vs[k], jnp.float32(0.0))


def _sc_mask(h):
    N = h.shape[0]
    info = plsc.get_sparse_core_info()
    nw = info.num_cores * info.num_subcores
    per_w = N // nw
    n_chunks = per_w // _CHUNK
    mesh = plsc.VectorSubcoreMesh(core_axis_name="c", subcore_axis_name="s")

    @functools.partial(
        pl.kernel, mesh=mesh,
        out_type=jax.ShapeDtypeStruct((N, _D), jnp.float32),
        scratch_types=[
            pltpu.VMEM((_CHUNK, _D), jnp.float32),
            pltpu.VMEM((_CHUNK, _D), jnp.float32),
        ],
    )
    def k(h_hbm, out_hbm, buf, obuf):
        wid = lax.axis_index("s") * info.num_cores + lax.axis_index("c")
        base = wid * per_w

        def chunk_body(j, carry):
            off = base + j * _CHUNK
            pltpu.sync_copy(h_hbm.at[pl.ds(off, _CHUNK)], buf)

            def tok_body(t, c2):
                _mask_one_token(buf, obuf, t)
                return c2

            lax.fori_loop(0, _CHUNK, tok_body, 0)
            pltpu.sync_copy(obuf, out_hbm.at[pl.ds(off, _CHUNK)])
            return carry

        lax.fori_loop(0, n_chunks, chunk_body, 0)

    return k(h)


def kernel(x, W, b):
    B, S, Dm = x.shape
    N = B * S
    x2 = x.reshape(N, Dm)
    h = _tc_matmul(x2, W, b.reshape(1, Dm))
    out = _sc_mask(h)
    return out.reshape(B, S, Dm)


# speedup vs baseline: 4.5738x; 4.5738x over previous
import jax
import jax.numpy as jnp
from jax.experimental import pallas as pl
from jax.experimental.pallas import tpu as pltpu

_NUM_HEADS = 12
_HEAD_DIM = 64
_TOPK = 8
_BLOCK_T = 4096


def _fused_body(x_ref, w_ref, b_ref, o_ref):
    xb = x_ref[...]
    acc = jax.lax.dot_general(
        w_ref[...], xb,
        dimension_numbers=(((1,), (1,)), ((), ())),
        preferred_element_type=jnp.float32,
    )
    h = acc + b_ref[...]
    neg = jnp.float32(-jnp.inf)
    heads = [h[i * _HEAD_DIM:(i + 1) * _HEAD_DIM, :] for i in range(_NUM_HEADS)]
    works = list(heads)
    for _ in range(_TOPK - 1):
        ms = [jnp.max(w, axis=0, keepdims=True) for w in works]
        works = [jnp.where(w == m, neg, w) for w, m in zip(works, ms)]
    thrs = [jnp.max(w, axis=0, keepdims=True) for w in works]
    parts = [jnp.where(g >= t, g, jnp.float32(0.0))
             for g, t in zip(heads, thrs)]
    masked = jnp.concatenate(parts, axis=0)
    o_ref[...] = masked.T


def kernel(x, W, b):
    B, S, Dm = x.shape
    N = B * S
    x2 = x.reshape(N, Dm)
    b2 = b.reshape(Dm, 1)
    T = _BLOCK_T
    grid = (N // T,)
    out = pl.pallas_call(
        _fused_body,
        grid=grid,
        in_specs=[
            pl.BlockSpec((T, Dm), lambda i: (i, 0)),
            pl.BlockSpec((Dm, Dm), lambda i: (0, 0)),
            pl.BlockSpec((Dm, 1), lambda i: (0, 0)),
        ],
        out_specs=pl.BlockSpec((T, Dm), lambda i: (i, 0)),
        out_shape=jax.ShapeDtypeStruct((N, Dm), jnp.float32),
        compiler_params=pltpu.CompilerParams(
            dimension_semantics=("arbitrary",),
        ),
    )(x2, W, b2)
    return out.reshape(B, S, Dm)
